# baseline (speedup 1.0000x reference)
import jax
import jax.numpy as jnp
from jax.experimental import pallas as pl
from jax.experimental.pallas import tpu as pltpu

_CHUNKS = (2000, 23000, 23500, 23500, 23000, 5000)
_MAX = max(_CHUNKS)


def _body(x_hbm, w_ref, b_ref, o_hbm, xb0, xb1, ob0, ob1, si0, si1, so0, so1):
    xbufs, obufs = (xb0, xb1), (ob0, ob1)
    isems, osems = (si0, si1), (so0, so1)
    offs = [0]
    for c in _CHUNKS:
        offs.append(offs[-1] + c)
    n = len(_CHUNKS)

    def copy_in(k):
        c = _CHUNKS[k]
        return pltpu.make_async_copy(
            x_hbm.at[pl.ds(offs[k], c), :],
            xbufs[k % 2].at[pl.ds(0, c), :],
            isems[k % 2],
        )

    def copy_out(k):
        c = _CHUNKS[k]
        return pltpu.make_async_copy(
            obufs[k % 2].at[pl.ds(0, c), :],
            o_hbm.at[pl.ds(offs[k], c), :],
            osems[k % 2],
        )

    copy_in(0).start()
    copy_in(1).start()
    for k in range(n):
        c = _CHUNKS[k]
        copy_in(k).wait()
        if k >= 2:
            copy_out(k - 2).wait()
        obufs[k % 2][pl.ds(0, c), :] = (
            jnp.dot(
                xbufs[k % 2][pl.ds(0, c), :],
                w_ref[...],
                preferred_element_type=jnp.float32,
            )
            + b_ref[...]
        )
        copy_out(k).start()
        if k + 2 < n:
            copy_in(k + 2).start()
    copy_out(n - 2).wait()
    copy_out(n - 1).wait()


def kernel(input, kernel, bias):
    n, in_ch = input.shape
    out_ch = kernel.shape[1]
    return pl.pallas_call(
        _body,
        in_specs=[
            pl.BlockSpec(memory_space=pl.ANY),
            pl.BlockSpec(memory_space=pltpu.MemorySpace.VMEM),
            pl.BlockSpec(memory_space=pltpu.MemorySpace.VMEM),
        ],
        out_specs=pl.BlockSpec(memory_space=pl.ANY),
        out_shape=jax.ShapeDtypeStruct((n, out_ch), jnp.float32),
        scratch_shapes=[
            pltpu.VMEM((_MAX, 128), jnp.float32),
            pltpu.VMEM((_MAX, 128), jnp.float32),
            pltpu.VMEM((_MAX, 128), jnp.float32),
            pltpu.VMEM((_MAX, 128), jnp.float32),
            pltpu.SemaphoreType.DMA,
            pltpu.SemaphoreType.DMA,
            pltpu.SemaphoreType.DMA,
            pltpu.SemaphoreType.DMA,
        ],
        compiler_params=pltpu.CompilerParams(
            vmem_limit_bytes=128 * 1024 * 1024,
        ),
    )(input, kernel, bias)


# speedup vs baseline: 1.2285x; 1.2285x over previous
import jax
import jax.numpy as jnp
from jax.experimental import pallas as pl
from jax.experimental.pallas import tpu as pltpu

_TILE = 28400


def _matmul_kernel(x_ref, w_ref, b_ref, o_ref):
    o_ref[...] = (
        jnp.dot(x_ref[...], w_ref[...], preferred_element_type=jnp.float32)
        + b_ref[...]
    )


def kernel(input, kernel, bias):
    n, in_ch = input.shape
    out_ch = kernel.shape[1]
    grid = (pl.cdiv(n, _TILE),)
    return pl.pallas_call(
        _matmul_kernel,
        grid=grid,
        in_specs=[
            pl.BlockSpec((_TILE, in_ch), lambda i: (i, 0)),
            pl.BlockSpec((in_ch, out_ch), lambda i: (0, 0)),
            pl.BlockSpec((1, out_ch), lambda i: (0, 0)),
        ],
        out_specs=pl.BlockSpec((_TILE, out_ch), lambda i: (i, 0)),
        out_shape=jax.ShapeDtypeStruct((n, out_ch), jnp.float32),
        compiler_params=pltpu.CompilerParams(
            dimension_semantics=("arbitrary",),
            vmem_limit_bytes=128 * 1024 * 1024,
        ),
    )(input, kernel, bias)
